# baseline (device time: 52502 ns/iter reference)
import jax
import jax.numpy as jnp
from jax import lax
from jax.experimental import pallas as pl
from jax.experimental.pallas import tpu as pltpu

N_DEV = 16
M = 2048
N = 2048
M_PER = M // N_DEV
HALF = N // 2
G2 = 2
GW = HALF // G2
N_HOP = N_DEV // 2

ST_DIRN = (0, 0, 1, 1)
ST_HALF = (0, 1, 1, 0)
ST_N = (8, 7, 8, 7)


def _gelu(y):
    c = 0.7978845608028654
    return 0.5 * y * (1.0 + jnp.tanh(c * (y + 0.044715 * y * y * y)))


def kernel(x, w_mat):
    def body(x_ref, w_ref, out_ref, part_ref, send_ref, recv_ref,
             send_sems, recv_sems):
        d = lax.axis_index("i")

        q = lax.rem(d, 4)
        z = lax.div(d, 4)
        p = jnp.where(q == 0, z,
            jnp.where(q == 1, 7 - z,
            jnp.where(q == 2, 8 + z, 15 - z)))

        def ring_dev(pos):
            pos = lax.rem(pos + 2 * N_DEV, N_DEV)
            c = lax.div(pos, 4)
            r = lax.rem(pos, 4)
            zz = jnp.where(lax.rem(c, 2) == 1, 3 - r, r)
            return 4 * zz + c

        right = ring_dev(p + 1)
        left = ring_dev(p - 1)

        def dest(st, s):
            off = (8 - s, 7 - s, 8 + s, 9 + s)[st]
            return ring_dev(p + off)

        def chunk_cols(c, st, g):
            col0 = ST_HALF[st] * HALF + g * GW
            return part_ref[pl.ds(c * M_PER, M_PER), col0:col0 + GW]

        def rdma(s, st, g):
            return pltpu.make_async_remote_copy(
                src_ref=send_ref.at[s, st, g],
                dst_ref=recv_ref.at[s, st, g],
                send_sem=send_sems.at[s, st, g],
                recv_sem=recv_sems.at[s, st, g],
                device_id=(right if ST_DIRN[st] == 0 else left,),
                device_id_type=pl.DeviceIdType.MESH,
            )

        p8 = jnp.dot(
            x_ref[pl.ds(ring_dev(p + 8) * M_PER, M_PER), :], w_ref[...],
            preferred_element_type=jnp.float32,
        ).astype(jnp.bfloat16)
        p7 = jnp.dot(
            x_ref[pl.ds(ring_dev(p + 7) * M_PER, M_PER), :], w_ref[...],
            preferred_element_type=jnp.float32,
        ).astype(jnp.bfloat16)
        p9 = jnp.dot(
            x_ref[pl.ds(ring_dev(p + 9) * M_PER, M_PER), :], w_ref[...],
            preferred_element_type=jnp.float32,
        ).astype(jnp.bfloat16)
        hop0_src = (p8, p7, p8, p9)
        for st in (0, 2, 1, 3):
            for g in range(G2):
                col0 = ST_HALF[st] * HALF + g * GW
                send_ref[0, st, g] = hop0_src[st][:, col0:col0 + GW]

        barrier_sem = pltpu.get_barrier_semaphore()
        for nbr in (left, right):
            pl.semaphore_signal(
                barrier_sem, inc=1,
                device_id=(nbr,), device_id_type=pl.DeviceIdType.MESH,
            )
        pl.semaphore_wait(barrier_sem, 2)

        for st in (0, 2, 1, 3):
            for g in range(G2):
                rdma(0, st, g).start()

        part_ref[...] = jnp.dot(
            x_ref[...], w_ref[...], preferred_element_type=jnp.float32
        ).astype(jnp.bfloat16)

        for s in range(1, N_HOP):
            for st in (0, 2, 1, 3):
                if s >= ST_N[st]:
                    continue
                for g in range(G2):
                    rdma(s - 1, st, g).wait_recv()
                    send_ref[s, st, g] = (
                        recv_ref[s - 1, st, g]
                        + chunk_cols(dest(st, s), st, g)
                    )
                    rdma(s, st, g).start()

        for st_a, st_b in ((0, 3), (1, 2)):
            for g in range(G2):
                rdma(ST_N[st_a] - 1, st_a, g).wait_recv()
                rdma(ST_N[st_b] - 1, st_b, g).wait_recv()
                col0 = ST_HALF[st_a] * HALF + g * GW
                y = (recv_ref[ST_N[st_a] - 1, st_a, g].astype(jnp.float32)
                     + recv_ref[ST_N[st_b] - 1, st_b, g].astype(jnp.float32)
                     + chunk_cols(d, st_a, g).astype(jnp.float32))
                out_ref[:, col0:col0 + GW] = _gelu(y)

        for st in range(4):
            for s in range(ST_N[st]):
                for g in range(G2):
                    rdma(s, st, g).wait_send()

    return pl.pallas_call(
        body,
        out_shape=jax.ShapeDtypeStruct((M_PER, N), jnp.float32),
        in_specs=[
            pl.BlockSpec(memory_space=pltpu.VMEM),
            pl.BlockSpec(memory_space=pltpu.VMEM),
        ],
        out_specs=pl.BlockSpec(memory_space=pltpu.VMEM),
        scratch_shapes=[
            pltpu.VMEM((M, N), jnp.bfloat16),
            pltpu.VMEM((N_HOP, 4, G2, M_PER, GW), jnp.bfloat16),
            pltpu.VMEM((N_HOP, 4, G2, M_PER, GW), jnp.bfloat16),
            pltpu.SemaphoreType.DMA((N_HOP, 4, G2)),
            pltpu.SemaphoreType.DMA((N_HOP, 4, G2)),
        ],
        compiler_params=pltpu.CompilerParams(collective_id=0),
    )(x, w_mat)


# device time: 51961 ns/iter; 1.0104x vs baseline; 1.0104x over previous
import jax
import jax.numpy as jnp
from jax import lax
from jax.experimental import pallas as pl
from jax.experimental.pallas import tpu as pltpu

N_DEV = 16
M = 2048
N = 2048
M_PER = M // N_DEV
HALF = N // 2
G2 = 1
GW = HALF // G2
N_HOP = N_DEV // 2

ST_DIRN = (0, 0, 1, 1)
ST_HALF = (0, 1, 1, 0)
ST_N = (8, 7, 8, 7)


def _gelu(y):
    c = 0.7978845608028654
    return 0.5 * y * (1.0 + jnp.tanh(c * (y + 0.044715 * y * y * y)))


def kernel(x, w_mat):
    def body(x_ref, w_ref, out_ref, part_ref, send_ref, recv_ref,
             send_sems, recv_sems):
        d = lax.axis_index("i")

        q = lax.rem(d, 4)
        z = lax.div(d, 4)
        p = jnp.where(q == 0, z,
            jnp.where(q == 1, 7 - z,
            jnp.where(q == 2, 8 + z, 15 - z)))

        def ring_dev(pos):
            pos = lax.rem(pos + 2 * N_DEV, N_DEV)
            c = lax.div(pos, 4)
            r = lax.rem(pos, 4)
            zz = jnp.where(lax.rem(c, 2) == 1, 3 - r, r)
            return 4 * zz + c

        right = ring_dev(p + 1)
        left = ring_dev(p - 1)

        def dest(st, s):
            off = (8 - s, 7 - s, 8 + s, 9 + s)[st]
            return ring_dev(p + off)

        def chunk_cols(c, st, g):
            col0 = ST_HALF[st] * HALF + g * GW
            return part_ref[pl.ds(c * M_PER, M_PER), col0:col0 + GW]

        def rdma(s, st, g):
            return pltpu.make_async_remote_copy(
                src_ref=send_ref.at[s, st, g],
                dst_ref=recv_ref.at[s, st, g],
                send_sem=send_sems.at[s, st, g],
                recv_sem=recv_sems.at[s, st, g],
                device_id=(right if ST_DIRN[st] == 0 else left,),
                device_id_type=pl.DeviceIdType.MESH,
            )

        p8 = jnp.dot(
            x_ref[pl.ds(ring_dev(p + 8) * M_PER, M_PER), :], w_ref[...],
            preferred_element_type=jnp.float32,
        ).astype(jnp.bfloat16)
        p7 = jnp.dot(
            x_ref[pl.ds(ring_dev(p + 7) * M_PER, M_PER), :], w_ref[...],
            preferred_element_type=jnp.float32,
        ).astype(jnp.bfloat16)
        p9 = jnp.dot(
            x_ref[pl.ds(ring_dev(p + 9) * M_PER, M_PER), :], w_ref[...],
            preferred_element_type=jnp.float32,
        ).astype(jnp.bfloat16)
        hop0_src = (p8, p7, p8, p9)
        for st in (0, 2, 1, 3):
            for g in range(G2):
                col0 = ST_HALF[st] * HALF + g * GW
                send_ref[0, st, g] = hop0_src[st][:, col0:col0 + GW]

        barrier_sem = pltpu.get_barrier_semaphore()
        for nbr in (left, right):
            pl.semaphore_signal(
                barrier_sem, inc=1,
                device_id=(nbr,), device_id_type=pl.DeviceIdType.MESH,
            )
        pl.semaphore_wait(barrier_sem, 2)

        for st in (0, 2, 1, 3):
            for g in range(G2):
                rdma(0, st, g).start()

        part_ref[...] = jnp.dot(
            x_ref[...], w_ref[...], preferred_element_type=jnp.float32
        ).astype(jnp.bfloat16)

        for s in range(1, N_HOP):
            for st in (0, 2, 1, 3):
                if s >= ST_N[st]:
                    continue
                for g in range(G2):
                    rdma(s - 1, st, g).wait_recv()
                    send_ref[s, st, g] = (
                        recv_ref[s - 1, st, g]
                        + chunk_cols(dest(st, s), st, g)
                    )
                    rdma(s, st, g).start()

        for st_a, st_b in ((0, 3), (1, 2)):
            for g in range(G2):
                rdma(ST_N[st_a] - 1, st_a, g).wait_recv()
                rdma(ST_N[st_b] - 1, st_b, g).wait_recv()
                col0 = ST_HALF[st_a] * HALF + g * GW
                y = (recv_ref[ST_N[st_a] - 1, st_a, g].astype(jnp.float32)
                     + recv_ref[ST_N[st_b] - 1, st_b, g].astype(jnp.float32)
                     + chunk_cols(d, st_a, g).astype(jnp.float32))
                out_ref[:, col0:col0 + GW] = _gelu(y)

        for st in range(4):
            for s in range(ST_N[st]):
                for g in range(G2):
                    rdma(s, st, g).wait_send()

    return pl.pallas_call(
        body,
        out_shape=jax.ShapeDtypeStruct((M_PER, N), jnp.float32),
        in_specs=[
            pl.BlockSpec(memory_space=pltpu.VMEM),
            pl.BlockSpec(memory_space=pltpu.VMEM),
        ],
        out_specs=pl.BlockSpec(memory_space=pltpu.VMEM),
        scratch_shapes=[
            pltpu.VMEM((M, N), jnp.bfloat16),
            pltpu.VMEM((N_HOP, 4, G2, M_PER, GW), jnp.bfloat16),
            pltpu.VMEM((N_HOP, 4, G2, M_PER, GW), jnp.bfloat16),
            pltpu.SemaphoreType.DMA((N_HOP, 4, G2)),
            pltpu.SemaphoreType.DMA((N_HOP, 4, G2)),
        ],
        compiler_params=pltpu.CompilerParams(collective_id=0),
    )(x, w_mat)
